# Initial kernel scaffold; baseline (speedup 1.0000x reference)
import jax
import jax.numpy as jnp
from jax.experimental import pallas as pl


def kernel(user_tensor, item_tensor, id_embedding, v_feat, W1, b1, W2, b2):
    raise NotImplementedError("write your pallas kernel here")


# speedup vs baseline: 1.6293x; 1.6293x over previous
import functools

import jax
import jax.numpy as jnp
from jax import lax
from jax.experimental import pallas as pl
from jax.experimental.pallas import tpu as pltpu
from jax.experimental.pallas import tpu_sc as plsc

NUM_USER = 100000
NUM_ITEM = 100000
DIM_E = 64
DIM_FEAT = 128
NUM_NEG = 127
BATCH = 1024
K = 1 + NUM_NEG
N = BATCH * K
TEMP_VALUE = 2.0
NUM_SAMPLE = 0.5
CONTRASTIVE = 0.5

_MLP_BLK = 2000


def _mlp_body(v_ref, w1_ref, b1_ref, w2_ref, b2_ref, out_ref):
    x = v_ref[...]
    nrm = jnp.sqrt(jnp.sum(x * x, axis=1, keepdims=True))
    x = x / jnp.maximum(nrm, 1e-12)
    h = jnp.dot(x, w1_ref[...], preferred_element_type=jnp.float32) + b1_ref[...]
    h = jnp.where(h >= 0, h, 0.01 * h)
    y = jnp.dot(h, w2_ref[...], preferred_element_type=jnp.float32) + b2_ref[...]
    out_ref[...] = y


def _mlp(v_feat, W1, b1, W2, b2):
    grid = NUM_ITEM // _MLP_BLK
    return pl.pallas_call(
        _mlp_body,
        grid=(grid,),
        in_specs=[
            pl.BlockSpec((_MLP_BLK, DIM_FEAT), lambda i: (i, 0)),
            pl.BlockSpec((DIM_FEAT, 256), lambda i: (0, 0)),
            pl.BlockSpec((1, 256), lambda i: (0, 0)),
            pl.BlockSpec((256, DIM_E), lambda i: (0, 0)),
            pl.BlockSpec((1, DIM_E), lambda i: (0, 0)),
        ],
        out_specs=pl.BlockSpec((_MLP_BLK, DIM_E), lambda i: (i, 0)),
        out_shape=jax.ShapeDtypeStruct((NUM_ITEM, DIM_E), jnp.float32),
    )(v_feat, W1, b1.reshape(1, 256), W2, b2.reshape(1, DIM_E))


_NW = 32
_PW = N // _NW
_CH = 1024
_SUB = 128
_PP = BATCH // _NW


def _gather_body(emb_hbm, feat_hbm, uidx_hbm, iidx_hbm, fidx_hbm, pidx_hbm,
                 u_out, e_out, f_out, p_out, idx_v, rows_v, sem):
    wid = lax.axis_index("s") * 2 + lax.axis_index("c")
    base = wid * _PW

    def one_gather(idx_hbm, table_hbm, out_hbm):
        pltpu.sync_copy(idx_hbm.at[pl.ds(base, _PW)], idx_v)

        def chunk(ci, carry):
            off = ci * _CH
            handles = []
            for k in range(_CH // _SUB):
                handles.append(pltpu.async_copy(
                    table_hbm.at[idx_v.at[pl.ds(off + k * _SUB, _SUB)]],
                    rows_v.at[pl.ds(k * _SUB, _SUB)], sem))
            for h in handles:
                h.wait()
            pltpu.sync_copy(rows_v, out_hbm.at[pl.ds(base + off, _CH)])
            return carry

        lax.fori_loop(0, _PW // _CH, chunk, 0)

    one_gather(uidx_hbm, emb_hbm, u_out)
    one_gather(iidx_hbm, emb_hbm, e_out)
    one_gather(fidx_hbm, feat_hbm, f_out)

    pbase = wid * _PP
    pltpu.sync_copy(pidx_hbm.at[pl.ds(pbase, _PP)], idx_v.at[pl.ds(0, _PP)])
    pltpu.async_copy(emb_hbm.at[idx_v.at[pl.ds(0, _PP)]],
                     rows_v.at[pl.ds(0, _PP)], sem).wait()
    pltpu.sync_copy(rows_v.at[pl.ds(0, _PP)], p_out.at[pl.ds(pbase, _PP)])


def _gather(id_embedding, feature, uidx, iidx, fidx, pidx):
    mesh = plsc.VectorSubcoreMesh(core_axis_name="c", subcore_axis_name="s")
    return pl.kernel(
        _gather_body,
        out_type=(
            jax.ShapeDtypeStruct((N, DIM_E), jnp.float32),
            jax.ShapeDtypeStruct((N, DIM_E), jnp.float32),
            jax.ShapeDtypeStruct((N, DIM_E), jnp.float32),
            jax.ShapeDtypeStruct((BATCH, DIM_E), jnp.float32),
        ),
        mesh=mesh,
        scratch_types=[
            pltpu.VMEM((_PW,), jnp.int32),
            pltpu.VMEM((_CH, DIM_E), jnp.float32),
            pltpu.SemaphoreType.DMA,
        ],
        compiler_params=pltpu.CompilerParams(use_tc_tiling_on_sc=False),
    )(id_embedding, feature, uidx, iidx, fidx, pidx)


_FB = 8192
_FG = _FB // K


def _fin_body(u_ref, e_ref, f_ref, p_ref, m_ref, s1_ref, s2_ref, ru_ref, re_ref):
    step = pl.program_id(0)
    u = u_ref[...]
    e = e_ref[...]
    f = f_ref[...]
    m = m_ref[...]

    fn = f / jnp.maximum(jnp.sqrt(jnp.sum(f * f, axis=1, keepdims=True)), 1e-12)
    p = p_ref[...]
    pn = p / jnp.maximum(jnp.sqrt(jnp.sum(p * p, axis=1, keepdims=True)), 1e-12)

    row = lax.broadcasted_iota(jnp.int32, (_FB, 1), 0)
    grp = row // K
    gcol = lax.broadcasted_iota(jnp.int32, (_FB, _FG), 1)
    gmask = (grp == gcol)
    gm = gmask.astype(jnp.float32)

    s_all = lax.dot_general(fn, pn, (((1,), (1,)), ((), ())),
                            preferred_element_type=jnp.float32)
    d1 = jnp.sum(jnp.where(gmask, s_all, 0.0), axis=1, keepdims=True)
    e1 = jnp.exp(d1 * (1.0 / TEMP_VALUE))

    mix = jnp.where(m > 0, f, e)
    d2 = jnp.sum(u * mix, axis=1, keepdims=True)
    e2 = jnp.exp(d2 * (1.0 / TEMP_VALUE))

    posm = (lax.rem(row, K) == 0)
    tot1 = jnp.sum(gm * e1, axis=0)
    tot2 = jnp.sum(gm * e2, axis=0)
    pos1 = jnp.sum(gm * jnp.where(posm, e1, 0.0), axis=0)
    pos2 = jnp.sum(gm * jnp.where(posm, e2, 0.0), axis=0)

    s1_ref[step, :] = -jnp.log(pos1 / tot1) * CONTRASTIVE
    s2_ref[step, :] = -jnp.log(pos2 / tot2) * (1.0 - CONTRASTIVE)

    su = jnp.sum(jnp.sqrt(jnp.sum(u * u, axis=1, keepdims=True)), axis=0, keepdims=True)
    se = jnp.sum(jnp.sqrt(jnp.sum(e * e, axis=1, keepdims=True)), axis=0, keepdims=True)

    @pl.when(step == 0)
    def _():
        ru_ref[...] = jnp.zeros_like(ru_ref)
        re_ref[...] = jnp.zeros_like(re_ref)

    ru_ref[...] += su
    re_ref[...] += se


def _finalize(u, e, f, p, maskf):
    grid = N // _FB
    return pl.pallas_call(
        _fin_body,
        grid=(grid,),
        in_specs=[
            pl.BlockSpec((_FB, DIM_E), lambda i: (i, 0)),
            pl.BlockSpec((_FB, DIM_E), lambda i: (i, 0)),
            pl.BlockSpec((_FB, DIM_E), lambda i: (i, 0)),
            pl.BlockSpec((_FG, DIM_E), lambda i: (i, 0)),
            pl.BlockSpec((_FB, 1), lambda i: (i, 0)),
        ],
        out_specs=[
            pl.BlockSpec((grid, _FG), lambda i: (0, 0)),
            pl.BlockSpec((grid, _FG), lambda i: (0, 0)),
            pl.BlockSpec((1, 1), lambda i: (0, 0)),
            pl.BlockSpec((1, 1), lambda i: (0, 0)),
        ],
        out_shape=(
            jax.ShapeDtypeStruct((grid, _FG), jnp.float32),
            jax.ShapeDtypeStruct((grid, _FG), jnp.float32),
            jax.ShapeDtypeStruct((1, 1), jnp.float32),
            jax.ShapeDtypeStruct((1, 1), jnp.float32),
        ),
    )(u, e, f, p, maskf)


def kernel(user_tensor, item_tensor, id_embedding, v_feat, W1, b1, W2, b2):
    feature = _mlp(v_feat, W1, b1, W2, b2)

    uidx = user_tensor.reshape(-1).astype(jnp.int32)
    iidx = item_tensor.reshape(-1).astype(jnp.int32)
    fidx = iidx - NUM_USER
    pidx = item_tensor[:, 0].astype(jnp.int32)

    u, e, f, p = _gather(id_embedding, feature, uidx, iidx, fidx, pidx)

    ridx = jax.random.randint(jax.random.key(42), (int(N * NUM_SAMPLE),), 0, N)
    maskf = jnp.zeros((N, 1), jnp.float32).at[ridx].set(1.0)

    s1m, s2m, ru, re = _finalize(u, e, f, p, maskf)
    reg = (ru[0, 0] + re[0, 0]) / (2.0 * N)
    return (s1m.reshape(-1), s2m.reshape(-1), reg)


# speedup vs baseline: 2.1350x; 1.3104x over previous
import functools

import jax
import jax.numpy as jnp
from jax import lax
from jax.experimental import pallas as pl
from jax.experimental.pallas import tpu as pltpu
from jax.experimental.pallas import tpu_sc as plsc

NUM_USER = 100000
NUM_ITEM = 100000
DIM_E = 64
DIM_FEAT = 128
NUM_NEG = 127
BATCH = 1024
K = 1 + NUM_NEG
N = BATCH * K
TEMP_VALUE = 2.0
NUM_SAMPLE = 0.5
CONTRASTIVE = 0.5

_MLP_BLK = 2000


def _mlp_body(v_ref, w1_ref, b1_ref, w2_ref, b2_ref, out_ref):
    x = v_ref[...]
    nrm = jnp.sqrt(jnp.sum(x * x, axis=1, keepdims=True))
    x = x / jnp.maximum(nrm, 1e-12)
    h = jnp.dot(x, w1_ref[...], preferred_element_type=jnp.float32) + b1_ref[...]
    h = jnp.where(h >= 0, h, 0.01 * h)
    y = jnp.dot(h, w2_ref[...], preferred_element_type=jnp.float32) + b2_ref[...]
    out_ref[...] = y


def _mlp(v_feat, W1, b1, W2, b2):
    grid = NUM_ITEM // _MLP_BLK
    return pl.pallas_call(
        _mlp_body,
        grid=(grid,),
        in_specs=[
            pl.BlockSpec((_MLP_BLK, DIM_FEAT), lambda i: (i, 0)),
            pl.BlockSpec((DIM_FEAT, 256), lambda i: (0, 0)),
            pl.BlockSpec((1, 256), lambda i: (0, 0)),
            pl.BlockSpec((256, DIM_E), lambda i: (0, 0)),
            pl.BlockSpec((1, DIM_E), lambda i: (0, 0)),
        ],
        out_specs=pl.BlockSpec((_MLP_BLK, DIM_E), lambda i: (i, 0)),
        out_shape=jax.ShapeDtypeStruct((NUM_ITEM, DIM_E), jnp.float32),
    )(v_feat, W1, b1.reshape(1, 256), W2, b2.reshape(1, DIM_E))


_NW = 32
_PW = N // _NW
_CH = 1024
_SUB = 128
_PP = BATCH // _NW


def _gather_body(emb_hbm, feat_hbm, uidx_hbm, iidx_hbm, fidx_hbm, pidx_hbm,
                 u_out, e_out, f_out, p_out, idx_v, rows_v, sem):
    wid = lax.axis_index("s") * 2 + lax.axis_index("c")
    base = wid * _PW

    def one_gather(idx_hbm, table_hbm, out_hbm):
        pltpu.sync_copy(idx_hbm.at[pl.ds(base, _PW)], idx_v)

        def chunk(ci, carry):
            off = ci * _CH
            handles = []
            for k in range(_CH // _SUB):
                handles.append(pltpu.async_copy(
                    table_hbm.at[idx_v.at[pl.ds(off + k * _SUB, _SUB)]],
                    rows_v.at[pl.ds(k * _SUB, _SUB)], sem))
            for h in handles:
                h.wait()
            pltpu.sync_copy(rows_v, out_hbm.at[pl.ds(base + off, _CH)])
            return carry

        lax.fori_loop(0, _PW // _CH, chunk, 0)

    one_gather(uidx_hbm, emb_hbm, u_out)
    one_gather(iidx_hbm, emb_hbm, e_out)
    one_gather(fidx_hbm, feat_hbm, f_out)

    pbase = wid * _PP
    pltpu.sync_copy(pidx_hbm.at[pl.ds(pbase, _PP)], idx_v.at[pl.ds(0, _PP)])
    pltpu.async_copy(emb_hbm.at[idx_v.at[pl.ds(0, _PP)]],
                     rows_v.at[pl.ds(0, _PP)], sem).wait()
    pltpu.sync_copy(rows_v.at[pl.ds(0, _PP)], p_out.at[pl.ds(pbase, _PP)])


def _gather(id_embedding, feature, uidx, iidx, fidx, pidx):
    mesh = plsc.VectorSubcoreMesh(core_axis_name="c", subcore_axis_name="s")
    return pl.kernel(
        _gather_body,
        out_type=(
            jax.ShapeDtypeStruct((N, DIM_E), jnp.float32),
            jax.ShapeDtypeStruct((N, DIM_E), jnp.float32),
            jax.ShapeDtypeStruct((N, DIM_E), jnp.float32),
            jax.ShapeDtypeStruct((BATCH, DIM_E), jnp.float32),
        ),
        mesh=mesh,
        scratch_types=[
            pltpu.VMEM((_PW,), jnp.int32),
            pltpu.VMEM((_CH, DIM_E), jnp.float32),
            pltpu.SemaphoreType.DMA,
        ],
        compiler_params=pltpu.CompilerParams(use_tc_tiling_on_sc=False),
    )(id_embedding, feature, uidx, iidx, fidx, pidx)


_FB = 8192
_FG = _FB // K


def _fin_body(u_ref, e_ref, f_ref, p_ref, m_ref, s1_ref, s2_ref, ru_ref, re_ref):
    step = pl.program_id(0)
    u = u_ref[...]
    e = e_ref[...]
    f = f_ref[...]
    m = m_ref[...]

    fn = f / jnp.maximum(jnp.sqrt(jnp.sum(f * f, axis=1, keepdims=True)), 1e-12)
    p = p_ref[...]
    pn = p / jnp.maximum(jnp.sqrt(jnp.sum(p * p, axis=1, keepdims=True)), 1e-12)

    row = lax.broadcasted_iota(jnp.int32, (_FB, 1), 0)
    grp = row // K
    gcol = lax.broadcasted_iota(jnp.int32, (_FB, _FG), 1)
    gmask = (grp == gcol)
    gm = gmask.astype(jnp.float32)

    s_all = lax.dot_general(fn, pn, (((1,), (1,)), ((), ())),
                            preferred_element_type=jnp.float32)
    d1 = jnp.sum(jnp.where(gmask, s_all, 0.0), axis=1, keepdims=True)
    e1 = jnp.exp(d1 * (1.0 / TEMP_VALUE))

    mix = jnp.where(m > 0, f, e)
    d2 = jnp.sum(u * mix, axis=1, keepdims=True)
    e2 = jnp.exp(d2 * (1.0 / TEMP_VALUE))

    posm = (lax.rem(row, K) == 0)
    tot1 = jnp.sum(gm * e1, axis=0)
    tot2 = jnp.sum(gm * e2, axis=0)
    pos1 = jnp.sum(gm * jnp.where(posm, e1, 0.0), axis=0)
    pos2 = jnp.sum(gm * jnp.where(posm, e2, 0.0), axis=0)

    s1_ref[step, :] = -jnp.log(pos1 / tot1) * CONTRASTIVE
    s2_ref[step, :] = -jnp.log(pos2 / tot2) * (1.0 - CONTRASTIVE)

    su = jnp.sum(jnp.sqrt(jnp.sum(u * u, axis=1, keepdims=True)), axis=0, keepdims=True)
    se = jnp.sum(jnp.sqrt(jnp.sum(e * e, axis=1, keepdims=True)), axis=0, keepdims=True)

    @pl.when(step == 0)
    def _():
        ru_ref[...] = jnp.zeros_like(ru_ref)
        re_ref[...] = jnp.zeros_like(re_ref)

    ru_ref[...] += su
    re_ref[...] += se


def _finalize(u, e, f, p, maskf):
    grid = N // _FB
    return pl.pallas_call(
        _fin_body,
        grid=(grid,),
        in_specs=[
            pl.BlockSpec((_FB, DIM_E), lambda i: (i, 0)),
            pl.BlockSpec((_FB, DIM_E), lambda i: (i, 0)),
            pl.BlockSpec((_FB, DIM_E), lambda i: (i, 0)),
            pl.BlockSpec((_FG, DIM_E), lambda i: (i, 0)),
            pl.BlockSpec((_FB, 1), lambda i: (i, 0)),
        ],
        out_specs=[
            pl.BlockSpec((grid, _FG), lambda i: (0, 0)),
            pl.BlockSpec((grid, _FG), lambda i: (0, 0)),
            pl.BlockSpec((1, 1), lambda i: (0, 0)),
            pl.BlockSpec((1, 1), lambda i: (0, 0)),
        ],
        out_shape=(
            jax.ShapeDtypeStruct((grid, _FG), jnp.float32),
            jax.ShapeDtypeStruct((grid, _FG), jnp.float32),
            jax.ShapeDtypeStruct((1, 1), jnp.float32),
            jax.ShapeDtypeStruct((1, 1), jnp.float32),
        ),
    )(u, e, f, p, maskf)


def kernel(user_tensor, item_tensor, id_embedding, v_feat, W1, b1, W2, b2):
    feature = _mlp(v_feat, W1, b1, W2, b2)

    uidx = user_tensor.reshape(-1).astype(jnp.int32)
    iidx = item_tensor.reshape(-1).astype(jnp.int32)
    fidx = iidx - NUM_USER
    pidx = item_tensor[:, 0].astype(jnp.int32)

    u, e, f, p = _gather(id_embedding, feature, uidx, iidx, fidx, pidx)

    with jax.ensure_compile_time_eval():
        ridx = jax.random.randint(jax.random.key(42), (int(N * NUM_SAMPLE),), 0, N)
        maskf = jnp.zeros((N, 1), jnp.float32).at[ridx].set(1.0)

    s1m, s2m, ru, re = _finalize(u, e, f, p, maskf)
    reg = (ru[0, 0] + re[0, 0]) / (2.0 * N)
    return (s1m.reshape(-1), s2m.reshape(-1), reg)
